# baseline (speedup 1.0000x reference)
import jax
import jax.numpy as jnp
from jax import lax
from jax.experimental import pallas as pl
from jax.experimental.pallas import tpu as pltpu
from jax.experimental.pallas import tpu_sc as plsc

B = 4096
T = 200
D = 64
VOCAB = 1000000

NC = 2
NS = 16
TR = T // 8
CB = B // 128

CW = 128
VFULL = (VOCAB // CW) * CW
NCHUNK = VFULL // CW
TAIL = VOCAB - VFULL
VPERM = VOCAB + TAIL
OUTROWS = VPERM // 2


def _ka_body(embt_hbm, tail_hbm, out_hbm, ib, sbuf, cbuf, tail_v,
             rsem0, rsem1, wsem0, wsem1):
    wid = lax.axis_index("s") * NC + lax.axis_index("c")
    rsems = (rsem0, rsem1)
    wsems = (wsem0, wsem1)

    iota = lax.iota(jnp.int32, 16)

    def read_copies(chunk, slot):
        c0 = chunk * CW
        return [
            pltpu.make_async_copy(
                embt_hbm.at[pl.ds(8 * r, 8), pl.ds(c0, CW)],
                ib.at[slot, r, :, pl.ds(0, CW)],
                rsems[slot],
            )
            for r in range(8)
        ]

    def write_copies(chunk, slot):
        w0 = chunk * (CW // 2)
        return [
            pltpu.make_async_copy(
                cbuf.at[slot],
                out_hbm.at[pl.ds(w0, CW // 2)],
                wsems[slot],
            )
        ]

    for cp in read_copies(wid, 0):
        cp.start()

    def nslab(i2, carry):
        for b in range(2):
            i = 2 * i2 + b
            chunk = i * 32 + wid

            @pl.when(chunk < NCHUNK)
            def _():
                nxt = chunk + 32

                @pl.when(nxt < NCHUNK)
                def _():
                    for cp in read_copies(nxt, 1 - b):
                        cp.start()

                for cp in read_copies(chunk, b):
                    cp.wait()

                @pl.when(i >= 2)
                def _():
                    for cp in write_copies(chunk - 64, b):
                        cp.wait()

                sb = sbuf.at[b]

                @plsc.parallel_loop(0, CW // 16, 1, unroll=2)
                def _(g):
                    j0 = 16 * g
                    wv = iota + lax.bitwise_and(j0, 63)
                    hv = lax.bitwise_and(lax.shift_right_logical(j0, 6), 1) * 64
                    for r in range(8):
                        for s in range(8):
                            d = 8 * r + s
                            val = ib[b, r, s, pl.ds(j0, 16)]
                            plsc.store_scatter(
                                sb, [wv, jnp.full((16,), d, jnp.int32) + hv],
                                val)

                cb = cbuf.at[b]

                @plsc.parallel_loop(0, CW // 2, 1, unroll=4)
                def _(w):
                    for h in range(8):
                        cb[w, pl.ds(16 * h, 16)] = sb[w, pl.ds(16 * h, 16)]

                for cp in write_copies(chunk, b):
                    cp.start()

        return carry

    niter = (NCHUNK + 31) // 32
    lax.fori_loop(0, (niter + 1) // 2, nslab, 0)

    li = lax.shift_right_logical(NCHUNK - 1 - wid, 5)
    for b in range(2):
        i_b = li - lax.bitwise_and(lax.bitwise_xor(li, b), 1)
        chunk_b = i_b * 32 + wid
        for cp in write_copies(chunk_b, b):
            cp.wait()

    @pl.when(wid == 0)
    def _():
        pltpu.sync_copy(tail_hbm, tail_v)
        pltpu.sync_copy(tail_v, out_hbm.at[pl.ds(VFULL // 2, TAIL)])


def _sc_body(tok_hbm, table_hbm, pos_hbm, out_hbm,
             pos_v, idx2, grow2, sbuf2, gsem0, gsem1, osem0, osem1):
    wid = lax.axis_index("s") * NC + lax.axis_index("c")
    gsems = (gsem0, gsem1)
    osems = (osem0, osem1)

    pltpu.sync_copy(pos_hbm, pos_v)

    iotas = [lax.iota(jnp.int32, 16) + 16 * q for q in range(4)]

    def start_gather(t, slot):
        tr = lax.shift_right_logical(t, 3)
        s = lax.bitwise_and(t, 7)
        pltpu.sync_copy(tok_hbm.at[tr, wid, s], idx2.at[slot])
        pltpu.make_async_copy(
            table_hbm.at[idx2.at[slot]], grow2.at[slot], gsems[slot]
        ).start()

    def wait_gather(slot):
        pltpu.make_async_copy(
            table_hbm.at[idx2.at[slot]], grow2.at[slot], gsems[slot]
        ).wait()

    def out_copy(t, r, slot):
        return pltpu.make_async_copy(
            sbuf2.at[slot, pl.ds(8 * r, 8), pl.ds(0, 128)],
            out_hbm.at[t, r, wid],
            osems[slot],
        )

    start_gather(0, 0)

    def gbody(g, carry):
        for b in range(2):
            t = 2 * g + b
            nt = t + 1

            @pl.when(nt < T)
            def _():
                start_gather(nt, 1 - b)

            wait_gather(b)

            @pl.when(t >= 2)
            def _():
                for r in range(8):
                    out_copy(t - 2, r, b).wait()

            pvec = [pos_v[t, pl.ds(16 * q, 16)] for q in range(4)]
            sb = sbuf2.at[b]

            @plsc.parallel_loop(0, 128, 1, unroll=8)
            def _(j):
                jf = jnp.full((16,), 0, jnp.int32) + j
                for q in range(4):
                    val = grow2[b, j, pl.ds(16 * q, 16)] + pvec[q]
                    plsc.store_scatter(sb, [iotas[q], jf], val)

            for r in range(8):
                out_copy(t, r, b).start()
        return carry

    lax.fori_loop(0, T // 2, gbody, 0)
    for b, t in ((0, T - 2), (1, T - 1)):
        for r in range(8):
            out_copy(t, r, b).wait()


@jax.jit
def _sc_transpose(embt, tail):
    mesh = plsc.VectorSubcoreMesh(core_axis_name="c", subcore_axis_name="s")
    fn = pl.kernel(
        _ka_body,
        out_type=jax.ShapeDtypeStruct((OUTROWS, 2 * D), jnp.float32),
        mesh=mesh,
        scratch_types=[
            pltpu.VMEM((2, 8, 8, CW + 1), jnp.float32),
            pltpu.VMEM((2, CW // 2 + 1, 129), jnp.float32),
            pltpu.VMEM((2, CW // 2, 2 * D), jnp.float32),
            pltpu.VMEM((TAIL, 2 * D), jnp.float32),
            pltpu.SemaphoreType.DMA,
            pltpu.SemaphoreType.DMA,
            pltpu.SemaphoreType.DMA,
            pltpu.SemaphoreType.DMA,
        ],
        compiler_params=pltpu.CompilerParams(
            use_tc_tiling_on_sc=True, needs_layout_passes=False
        ),
    )
    return fn(embt, tail)


@jax.jit
def _sc_lookup(tok5, emb_weight, pos):
    mesh = plsc.VectorSubcoreMesh(core_axis_name="c", subcore_axis_name="s")
    fn = pl.kernel(
        _sc_body,
        out_type=jax.ShapeDtypeStruct((T, 8, CB, 8, 128), jnp.float32),
        mesh=mesh,
        scratch_types=[
            pltpu.VMEM((T, D), jnp.float32),
            pltpu.VMEM((2, 128), jnp.int32),
            pltpu.VMEM((2, 128, D), jnp.float32),
            pltpu.VMEM((2, D, 133), jnp.float32),
            pltpu.SemaphoreType.DMA,
            pltpu.SemaphoreType.DMA,
            pltpu.SemaphoreType.DMA,
            pltpu.SemaphoreType.DMA,
        ],
        compiler_params=pltpu.CompilerParams(
            use_tc_tiling_on_sc=False, needs_layout_passes=False
        ),
    )
    return fn(tok5, emb_weight, pos)


def kernel(tokens, emb_weight, pos):
    tok = tokens.astype(jnp.int32)
    tokp = ((tok & -128) | ((tok & 63) << 1) | ((tok >> 6) & 1))
    tok5 = tokp.reshape(CB, 128, TR, 8).transpose(2, 0, 3, 1)
    tail = jnp.pad(emb_weight[VFULL:], ((0, 0), (0, D)))
    table2 = _sc_transpose(emb_weight.T, tail)
    out5 = _sc_lookup(tok5, table2.reshape(VPERM, D), pos)
    return out5.transpose(2, 4, 0, 1, 3).reshape(B, T, D)


# speedup vs baseline: 1.5317x; 1.5317x over previous
import jax
import jax.numpy as jnp
from jax import lax
from jax.experimental import pallas as pl
from jax.experimental.pallas import tpu as pltpu
from jax.experimental.pallas import tpu_sc as plsc

B = 4096
T = 200
D = 64
VOCAB = 1000000

NC = 2
NS = 16
TR = T // 8
CB = B // 128


def _sc_body(tok_hbm, table_hbm, pos_hbm, out_hbm,
             pos_v, idx2, grow2, sbuf2, gsem0, gsem1, osem0, osem1):
    wid = lax.axis_index("s") * NC + lax.axis_index("c")
    gsems = (gsem0, gsem1)
    osems = (osem0, osem1)

    pltpu.sync_copy(pos_hbm, pos_v)

    iotas = [lax.iota(jnp.int32, 16) + 16 * q for q in range(4)]

    def start_gather(t, slot):
        tr = lax.shift_right_logical(t, 3)
        s = lax.bitwise_and(t, 7)
        pltpu.sync_copy(tok_hbm.at[tr, wid, s], idx2.at[slot])
        pltpu.make_async_copy(
            table_hbm.at[idx2.at[slot]], grow2.at[slot], gsems[slot]
        ).start()

    def wait_gather(slot):
        pltpu.make_async_copy(
            table_hbm.at[idx2.at[slot]], grow2.at[slot], gsems[slot]
        ).wait()

    def out_copy(t, r, slot):
        return pltpu.make_async_copy(
            sbuf2.at[slot, pl.ds(8 * r, 8), pl.ds(0, 128)],
            out_hbm.at[t, r, wid],
            osems[slot],
        )

    start_gather(0, 0)

    def gbody(g, carry):
        for b in range(2):
            t = 2 * g + b
            nt = t + 1

            @pl.when(nt < T)
            def _():
                start_gather(nt, 1 - b)

            wait_gather(b)

            @pl.when(t >= 2)
            def _():
                for r in range(8):
                    out_copy(t - 2, r, b).wait()

            pvec = [pos_v[t, pl.ds(16 * q, 16)] for q in range(4)]
            sb = sbuf2.at[b]

            @plsc.parallel_loop(0, 128, 1, unroll=8)
            def _(j):
                jf = jnp.full((16,), 0, jnp.int32) + j
                for q in range(4):
                    val = grow2[b, j, pl.ds(16 * q, 16)] + pvec[q]
                    plsc.store_scatter(sb, [iotas[q], jf], val)

            for r in range(8):
                out_copy(t, r, b).start()
        return carry

    lax.fori_loop(0, T // 2, gbody, 0)
    for b, t in ((0, T - 2), (1, T - 1)):
        for r in range(8):
            out_copy(t, r, b).wait()


@jax.jit
def _sc_lookup(tok5, emb_weight, pos):
    mesh = plsc.VectorSubcoreMesh(core_axis_name="c", subcore_axis_name="s")
    fn = pl.kernel(
        _sc_body,
        out_type=jax.ShapeDtypeStruct((T, 8, CB, 8, 128), jnp.float32),
        mesh=mesh,
        scratch_types=[
            pltpu.VMEM((T, D), jnp.float32),
            pltpu.VMEM((2, 128), jnp.int32),
            pltpu.VMEM((2, 128, D), jnp.float32),
            pltpu.VMEM((2, D, 133), jnp.float32),
            pltpu.SemaphoreType.DMA,
            pltpu.SemaphoreType.DMA,
            pltpu.SemaphoreType.DMA,
            pltpu.SemaphoreType.DMA,
        ],
        compiler_params=pltpu.CompilerParams(
            use_tc_tiling_on_sc=False, needs_layout_passes=False
        ),
    )
    return fn(tok5, emb_weight, pos)


def kernel(tokens, emb_weight, pos):
    tok5 = (tokens.astype(jnp.int32)
            .reshape(CB, 128, TR, 8).transpose(2, 0, 3, 1))
    out5 = _sc_lookup(tok5, emb_weight, pos)
    return out5.transpose(2, 4, 0, 1, 3).reshape(B, T, D)


# speedup vs baseline: 1.6103x; 1.0513x over previous
import jax
import jax.numpy as jnp
from jax import lax
from jax.experimental import pallas as pl
from jax.experimental.pallas import tpu as pltpu
from jax.experimental.pallas import tpu_sc as plsc

B = 4096
T = 200
D = 64
VOCAB = 1000000

NC = 2
NS = 16
TR = T // 8
CB = B // 128


def _sc_body(tok_hbm, table_hbm, pos_hbm, out_hbm,
             pos_v, idx2, grow2, sbuf2, gsem0, gsem1, osem0, osem1):
    wid = lax.axis_index("s") * NC + lax.axis_index("c")
    gsems = (gsem0, gsem1)
    osems = (osem0, osem1)

    pltpu.sync_copy(pos_hbm, pos_v)

    iotas = [lax.iota(jnp.int32, 16) + 16 * q for q in range(4)]

    def start_gather(t, slot):
        tr = lax.shift_right_logical(t, 3)
        s = lax.bitwise_and(t, 7)
        pltpu.sync_copy(tok_hbm.at[tr, wid, s], idx2.at[slot])
        pltpu.make_async_copy(
            table_hbm.at[idx2.at[slot]], grow2.at[slot], gsems[slot]
        ).start()

    def wait_gather(slot):
        pltpu.make_async_copy(
            table_hbm.at[idx2.at[slot]], grow2.at[slot], gsems[slot]
        ).wait()

    def out_copy(t, r, slot):
        return pltpu.make_async_copy(
            sbuf2.at[slot, pl.ds(8 * r, 8), pl.ds(0, 128)],
            out_hbm.at[t, r, wid],
            osems[slot],
        )

    start_gather(0, 0)

    def gbody(g, carry):
        for b in range(2):
            t = 2 * g + b
            nt = t + 1

            @pl.when(nt < T)
            def _():
                start_gather(nt, 1 - b)

            wait_gather(b)

            @pl.when(t >= 2)
            def _():
                for r in range(8):
                    out_copy(t - 2, r, b).wait()

            pvec = [pos_v[t, pl.ds(16 * q, 16)] for q in range(4)]
            sb = sbuf2.at[b]

            @plsc.parallel_loop(0, 128, 1, unroll=16)
            def _(j):
                jf = jnp.full((16,), 0, jnp.int32) + j
                for q in range(4):
                    val = grow2[b, j, pl.ds(16 * q, 16)] + pvec[q]
                    plsc.store_scatter(sb, [iotas[q], jf], val)

            for r in range(8):
                out_copy(t, r, b).start()
        return carry

    lax.fori_loop(0, T // 2, gbody, 0)
    for b, t in ((0, T - 2), (1, T - 1)):
        for r in range(8):
            out_copy(t, r, b).wait()


@jax.jit
def _sc_lookup(tok5, emb_weight, pos):
    mesh = plsc.VectorSubcoreMesh(core_axis_name="c", subcore_axis_name="s")
    fn = pl.kernel(
        _sc_body,
        out_type=jax.ShapeDtypeStruct((T, 8, CB, 8, 128), jnp.float32),
        mesh=mesh,
        scratch_types=[
            pltpu.VMEM((T, D), jnp.float32),
            pltpu.VMEM((2, 128), jnp.int32),
            pltpu.VMEM((2, 128, D), jnp.float32),
            pltpu.VMEM((2, D, 133), jnp.float32),
            pltpu.SemaphoreType.DMA,
            pltpu.SemaphoreType.DMA,
            pltpu.SemaphoreType.DMA,
            pltpu.SemaphoreType.DMA,
        ],
        compiler_params=pltpu.CompilerParams(
            use_tc_tiling_on_sc=False, needs_layout_passes=False
        ),
    )
    return fn(tok5, emb_weight, pos)


def kernel(tokens, emb_weight, pos):
    tok5 = (tokens.astype(jnp.int32)
            .reshape(CB, 128, TR, 8).transpose(2, 0, 3, 1))
    out5 = _sc_lookup(tok5, emb_weight, pos)
    return out5.transpose(2, 4, 0, 1, 3).reshape(B, T, D)
